# baseline (speedup 1.0000x reference)
import functools

import jax
import jax.numpy as jnp
from jax.experimental import pallas as pl
from jax.experimental.pallas import tpu as pltpu

N = 1024
HID = 64
HEADS = 4
C = 16
BLK = 128
GRID = N // BLK
NEG = -3e38


def _nt(a, b):
    return jax.lax.dot_general(a, b, (((1,), (1,)), ((), ())),
                               precision=jax.lax.Precision.HIGHEST,
                               preferred_element_type=jnp.float32)


def _nn(a, b):
    return jax.lax.dot_general(a, b, (((1,), (0,)), ((), ())),
                               precision=jax.lax.Precision.HIGHEST,
                               preferred_element_type=jnp.float32)


def _relu(v):
    return jnp.maximum(v, 0.0)


def _attbd(att_row):
    head = jax.lax.broadcasted_iota(jnp.int32, (8, HID), 0)
    lane = jax.lax.broadcasted_iota(jnp.int32, (8, HID), 1)
    return jnp.where(lane // C == head, att_row, 0.0)


def _mask_body(ei_ref, cntt_ref):
    i0 = pl.program_id(0) * BLK
    lane = jax.lax.broadcasted_iota(jnp.int32, (BLK, N), 1)
    row = jax.lax.broadcasted_iota(jnp.int32, (BLK, N), 0) + i0
    m = (ei_ref[...] != 0).astype(jnp.float32)
    cnt = m + (lane == row).astype(jnp.float32)
    cntt_ref[...] = jnp.transpose(cnt)


def _prep1_body(x_ref, wp1_ref, bp1_ref, wp2_ref, bp2_ref,
                wl_ref, blc_ref, wr_ref, br_ref, attbd_ref,
                h_ref, xlt_ref, xr_ref, a_ref, b_ref):
    h0 = _relu(_nt(x_ref[...], wp1_ref[...]) + bp1_ref[...])
    h1 = h0 + _relu(_nt(h0, wp2_ref[...]) + bp2_ref[...])
    xlt = _nt(wl_ref[...], h1) + blc_ref[...]
    xr = _nt(h1, wr_ref[...]) + br_ref[...]
    attbd = _attbd(attbd_ref[...]) * 0.2
    h_ref[...] = h1
    xlt_ref[...] = xlt
    xr_ref[...] = xr
    a_ref[...] = _nn(attbd, xlt)
    b_ref[...] = _nt(xr, attbd)


def _prep2_body(h_ref, g_ref, wp_ref, bp_ref,
                wl_ref, blc_ref, wr_ref, br_ref, attbd_ref,
                hn_ref, xlt_ref, xr_ref, a_ref, b_ref):
    h2 = h_ref[...] + g_ref[...]
    h3 = h2 + _relu(_nt(h2, wp_ref[...]) + bp_ref[...])
    xlt = _nt(wl_ref[...], h3) + blc_ref[...]
    xr = _nt(h3, wr_ref[...]) + br_ref[...]
    attbd = _attbd(attbd_ref[...]) * 0.2
    hn_ref[...] = h3
    xlt_ref[...] = xlt
    xr_ref[...] = xr
    a_ref[...] = _nn(attbd, xlt)
    b_ref[...] = _nt(xr, attbd)


def _prep3_body(h_ref, g_ref, wp5_ref, bp5_ref, we1_ref, be1_ref,
                we2_ref, be2_ref, p_ref, qt_ref):
    h4 = h_ref[...] + g_ref[...]
    h5 = h4 + _relu(_nt(h4, wp5_ref[...]) + bp5_ref[...])
    w12 = _nn(we2_ref[...], we1_ref[...])
    c12 = _nt(be1_ref[...], we2_ref[...]) + be2_ref[...]
    p_ref[...] = _nt(h5, w12[:, :HID]) + c12
    qt_ref[...] = _nt(w12[:, HID:], h5)


def _gat_body(xlt_ref, a_ref, att_ref, bc_ref, xr_ref, b_ref, cntt_ref,
              g_ref):
    xr = xr_ref[...]
    att = att_ref[...] * 0.8
    cnt = cntt_ref[...]
    valid = cnt > 0.0
    for h in range(HEADS):
        parts = []
        for s0 in range(0, N, 256):
            pacc = b_ref[:, h:h + 1] + a_ref[h:h + 1, s0:s0 + 256]
            for c in range(C):
                hc = h * C + c
                t = xr[:, hc:hc + 1] + xlt_ref[hc:hc + 1, s0:s0 + 256]
                pacc = pacc + att[0:1, hc:hc + 1] * _relu(t)
            parts.append(pacc)
        acc = jnp.concatenate(parts, axis=1)
        m = jnp.max(jnp.where(valid, acc, NEG), axis=1, keepdims=True)
        num = cnt * jnp.exp(jnp.minimum(acc - m, 0.0))
        den = jnp.sum(num, axis=1, keepdims=True) + 1e-16
        g_ref[:, h * C:(h + 1) * C] = (
            _nt(num, xlt_ref[h * C:(h + 1) * C, :]) / den
            + bc_ref[:, h * C:(h + 1) * C])


def _edge_body(p_ref, qt_ref, w3_ref, be3_ref, ei_ref, o_ref):
    i0 = pl.program_id(0) * BLK
    lane = jax.lax.broadcasted_iota(jnp.int32, (BLK, N), 1)
    row = jax.lax.broadcasted_iota(jnp.int32, (BLK, N), 0) + i0
    keep = (ei_ref[...] != 0) & (lane != row)
    p = p_ref[...]
    w3 = w3_ref[...]
    parts = []
    for s0 in range(0, N, 256):
        pacc = jnp.zeros((BLK, 256), jnp.float32) + be3_ref[0:1, 0:1]
        for c in range(HID):
            t = p[:, c:c + 1] + qt_ref[c:c + 1, s0:s0 + 256]
            pacc = pacc + w3[0:1, c:c + 1] * _relu(t)
        parts.append(pacc)
    acc = jnp.concatenate(parts, axis=1)
    o_ref[...] = jnp.where(keep, jax.nn.sigmoid(acc), 0.0)


def _full(shape):
    return pl.BlockSpec(shape, lambda *_: tuple(0 for _ in shape))


_F64 = jax.ShapeDtypeStruct((N, HID), jnp.float32)
_F64T = jax.ShapeDtypeStruct((HID, N), jnp.float32)
_A8 = jax.ShapeDtypeStruct((8, N), jnp.float32)
_B8 = jax.ShapeDtypeStruct((N, 8), jnp.float32)
_MB = jax.ShapeDtypeStruct((N, N), jnp.float32)

_maskprep = pl.pallas_call(
    _mask_body,
    grid=(GRID,),
    in_specs=[pl.BlockSpec((BLK, N), lambda i: (i, 0))],
    out_specs=pl.BlockSpec((N, BLK), lambda i: (0, i)),
    out_shape=_MB,
    compiler_params=pltpu.CompilerParams(
        dimension_semantics=("arbitrary",)),
)

_prep1 = pl.pallas_call(
    _prep1_body, out_shape=(_F64, _F64T, _F64, _A8, _B8))
_prep2 = pl.pallas_call(
    _prep2_body, out_shape=(_F64, _F64T, _F64, _A8, _B8))
_prep3 = pl.pallas_call(
    _prep3_body, out_shape=(_F64, _F64T))

_gat = pl.pallas_call(
    _gat_body,
    grid=(GRID,),
    in_specs=[
        _full((HID, N)),
        _full((8, N)),
        _full((1, HID)),
        _full((1, HID)),
        pl.BlockSpec((BLK, HID), lambda i: (i, 0)),
        pl.BlockSpec((BLK, 8), lambda i: (i, 0)),
        pl.BlockSpec((BLK, N), lambda i: (i, 0)),
    ],
    out_specs=pl.BlockSpec((BLK, HID), lambda i: (i, 0)),
    out_shape=_F64,
    compiler_params=pltpu.CompilerParams(
        dimension_semantics=("parallel",)),
)

_edge = pl.pallas_call(
    _edge_body,
    grid=(GRID,),
    in_specs=[
        pl.BlockSpec((BLK, HID), lambda i: (i, 0)),
        _full((HID, N)),
        _full((1, HID)),
        _full((1, 1)),
        pl.BlockSpec((BLK, N), lambda i: (i, 0)),
    ],
    out_specs=pl.BlockSpec((BLK, N), lambda i: (i, 0)),
    out_shape=jax.ShapeDtypeStruct((N, N), jnp.float32),
    compiler_params=pltpu.CompilerParams(
        dimension_semantics=("parallel",)),
)


@functools.partial(jax.jit, static_argnums=())
def kernel(x, edge_index, Wp1, bp1, Wp2, bp2, Wl1, bl1, Wr1, br1, att1,
           bc1, Wp4, bp4, Wl2, bl2, Wr2, br2, att2, bc2, Wp5, bp5,
           We1, be1, We2, be2, We3, be3):
    cntt = _maskprep(edge_index)

    def row(v):
        return v.reshape(1, -1)

    h1, xlt1, xr1, a1, b1 = _prep1(
        x, Wp1, row(bp1), Wp2, row(bp2),
        Wl1, bl1.reshape(HID, 1), Wr1, row(br1), row(att1))
    g1 = _gat(xlt1, a1, row(att1), row(bc1), xr1, b1, cntt)
    h3, xlt2, xr2, a2, b2 = _prep2(
        h1, g1, Wp4, row(bp4),
        Wl2, bl2.reshape(HID, 1), Wr2, row(br2), row(att2))
    g2 = _gat(xlt2, a2, row(att2), row(bc2), xr2, b2, cntt)
    p, qt = _prep3(h3, g2, Wp5, row(bp5), We1, row(be1), We2, row(be2))
    return _edge(p, qt, row(We3[0]), be3.reshape(1, 1), edge_index)


# speedup vs baseline: 1.0161x; 1.0161x over previous
import functools

import jax
import jax.numpy as jnp
from jax.experimental import pallas as pl
from jax.experimental.pallas import tpu as pltpu

N = 1024
HID = 64
HEADS = 4
C = 16
BLK = 128
GRID = N // BLK
NEG = -3e38


def _nt(a, b):
    return jax.lax.dot_general(a, b, (((1,), (1,)), ((), ())),
                               precision=jax.lax.Precision.HIGHEST,
                               preferred_element_type=jnp.float32)


def _nn(a, b):
    return jax.lax.dot_general(a, b, (((1,), (0,)), ((), ())),
                               precision=jax.lax.Precision.HIGHEST,
                               preferred_element_type=jnp.float32)


def _relu(v):
    return jnp.maximum(v, 0.0)


def _attbd(att_row):
    head = jax.lax.broadcasted_iota(jnp.int32, (8, HID), 0)
    lane = jax.lax.broadcasted_iota(jnp.int32, (8, HID), 1)
    return jnp.where(lane // C == head, att_row, 0.0)


def _mask_body(ei_ref, keep_ref, cntt_ref):
    i0 = pl.program_id(0) * BLK
    lane = jax.lax.broadcasted_iota(jnp.int32, (BLK, N), 1)
    row = jax.lax.broadcasted_iota(jnp.int32, (BLK, N), 0) + i0
    m = (ei_ref[...] != 0).astype(jnp.float32)
    diag = lane == row
    keep_ref[...] = jnp.where(diag, 0.0, m)
    cnt = m + diag.astype(jnp.float32)
    cntt_ref[...] = jnp.transpose(cnt)


def _prep1_body(x_ref, wp1_ref, bp1_ref, wp2_ref, bp2_ref,
                wl_ref, blc_ref, wr_ref, br_ref, attbd_ref,
                h_ref, xlt_ref, xr_ref, a_ref, b_ref):
    h0 = _relu(_nt(x_ref[...], wp1_ref[...]) + bp1_ref[...])
    h1 = h0 + _relu(_nt(h0, wp2_ref[...]) + bp2_ref[...])
    xlt = _nt(wl_ref[...], h1) + blc_ref[...]
    xr = _nt(h1, wr_ref[...]) + br_ref[...]
    attbd = _attbd(attbd_ref[...]) * 0.2
    h_ref[...] = h1
    xlt_ref[...] = xlt
    xr_ref[...] = xr
    a_ref[...] = _nn(attbd, xlt)
    b_ref[...] = _nt(xr, attbd)


def _prep2_body(h_ref, g_ref, wp_ref, bp_ref,
                wl_ref, blc_ref, wr_ref, br_ref, attbd_ref,
                hn_ref, xlt_ref, xr_ref, a_ref, b_ref):
    h2 = h_ref[...] + g_ref[...]
    h3 = h2 + _relu(_nt(h2, wp_ref[...]) + bp_ref[...])
    xlt = _nt(wl_ref[...], h3) + blc_ref[...]
    xr = _nt(h3, wr_ref[...]) + br_ref[...]
    attbd = _attbd(attbd_ref[...]) * 0.2
    hn_ref[...] = h3
    xlt_ref[...] = xlt
    xr_ref[...] = xr
    a_ref[...] = _nn(attbd, xlt)
    b_ref[...] = _nt(xr, attbd)


def _prep3_body(h_ref, g_ref, wp5_ref, bp5_ref, we1_ref, be1_ref,
                we2_ref, be2_ref, p_ref, qt_ref):
    h4 = h_ref[...] + g_ref[...]
    h5 = h4 + _relu(_nt(h4, wp5_ref[...]) + bp5_ref[...])
    w12 = _nn(we2_ref[...], we1_ref[...])
    c12 = _nt(be1_ref[...], we2_ref[...]) + be2_ref[...]
    p_ref[...] = _nt(h5, w12[:, :HID]) + c12
    qt_ref[...] = _nt(w12[:, HID:], h5)


def _gat_body(xlt_ref, a_ref, att_ref, bc_ref, xr_ref, b_ref, cntt_ref,
              g_ref):
    xr = xr_ref[...]
    att = att_ref[...] * 0.8
    cnt = cntt_ref[...]
    valid = cnt > 0.0
    for h in range(HEADS):
        parts = []
        for s0 in range(0, N, 256):
            pacc = b_ref[:, h:h + 1] + a_ref[h:h + 1, s0:s0 + 256]
            for c in range(C):
                hc = h * C + c
                t = xr[:, hc:hc + 1] + xlt_ref[hc:hc + 1, s0:s0 + 256]
                pacc = pacc + att[0:1, hc:hc + 1] * _relu(t)
            parts.append(pacc)
        acc = jnp.concatenate(parts, axis=1)
        m = jnp.max(jnp.where(valid, acc, NEG), axis=1, keepdims=True)
        num = cnt * jnp.exp(jnp.minimum(acc - m, 0.0))
        den = jnp.sum(num, axis=1, keepdims=True) + 1e-16
        g_ref[:, h * C:(h + 1) * C] = (
            _nt(num, xlt_ref[h * C:(h + 1) * C, :]) / den
            + bc_ref[:, h * C:(h + 1) * C])


def _edge_body(p_ref, qt_ref, w3_ref, be3_ref, keep_ref, o_ref):
    p = p_ref[...]
    w3 = w3_ref[...]
    parts = []
    for s0 in range(0, N, 256):
        pacc = jnp.zeros((BLK, 256), jnp.float32) + be3_ref[0:1, 0:1]
        for c in range(HID):
            t = p[:, c:c + 1] + qt_ref[c:c + 1, s0:s0 + 256]
            pacc = pacc + w3[0:1, c:c + 1] * _relu(t)
        parts.append(pacc)
    acc = jnp.concatenate(parts, axis=1)
    o_ref[...] = jax.nn.sigmoid(acc) * keep_ref[...]


def _full(shape):
    return pl.BlockSpec(shape, lambda *_: tuple(0 for _ in shape))


_F64 = jax.ShapeDtypeStruct((N, HID), jnp.float32)
_F64T = jax.ShapeDtypeStruct((HID, N), jnp.float32)
_A8 = jax.ShapeDtypeStruct((8, N), jnp.float32)
_B8 = jax.ShapeDtypeStruct((N, 8), jnp.float32)
_MB = jax.ShapeDtypeStruct((N, N), jnp.float32)

_maskprep = pl.pallas_call(
    _mask_body,
    grid=(GRID,),
    in_specs=[pl.BlockSpec((BLK, N), lambda i: (i, 0))],
    out_specs=(pl.BlockSpec((BLK, N), lambda i: (i, 0)),
               pl.BlockSpec((N, BLK), lambda i: (0, i))),
    out_shape=(_MB, _MB),
    compiler_params=pltpu.CompilerParams(
        dimension_semantics=("arbitrary",)),
)

_prep1 = pl.pallas_call(
    _prep1_body, out_shape=(_F64, _F64T, _F64, _A8, _B8))
_prep2 = pl.pallas_call(
    _prep2_body, out_shape=(_F64, _F64T, _F64, _A8, _B8))
_prep3 = pl.pallas_call(
    _prep3_body, out_shape=(_F64, _F64T))

_gat = pl.pallas_call(
    _gat_body,
    grid=(GRID,),
    in_specs=[
        _full((HID, N)),
        _full((8, N)),
        _full((1, HID)),
        _full((1, HID)),
        pl.BlockSpec((BLK, HID), lambda i: (i, 0)),
        pl.BlockSpec((BLK, 8), lambda i: (i, 0)),
        pl.BlockSpec((BLK, N), lambda i: (i, 0)),
    ],
    out_specs=pl.BlockSpec((BLK, HID), lambda i: (i, 0)),
    out_shape=_F64,
    compiler_params=pltpu.CompilerParams(
        dimension_semantics=("parallel",)),
)

_edge = pl.pallas_call(
    _edge_body,
    grid=(GRID,),
    in_specs=[
        pl.BlockSpec((BLK, HID), lambda i: (i, 0)),
        _full((HID, N)),
        _full((1, HID)),
        _full((1, 1)),
        pl.BlockSpec((BLK, N), lambda i: (i, 0)),
    ],
    out_specs=pl.BlockSpec((BLK, N), lambda i: (i, 0)),
    out_shape=jax.ShapeDtypeStruct((N, N), jnp.float32),
    compiler_params=pltpu.CompilerParams(
        dimension_semantics=("parallel",)),
)


@functools.partial(jax.jit, static_argnums=())
def kernel(x, edge_index, Wp1, bp1, Wp2, bp2, Wl1, bl1, Wr1, br1, att1,
           bc1, Wp4, bp4, Wl2, bl2, Wr2, br2, att2, bc2, Wp5, bp5,
           We1, be1, We2, be2, We3, be3):
    keep, cntt = _maskprep(edge_index)

    def row(v):
        return v.reshape(1, -1)

    h1, xlt1, xr1, a1, b1 = _prep1(
        x, Wp1, row(bp1), Wp2, row(bp2),
        Wl1, bl1.reshape(HID, 1), Wr1, row(br1), row(att1))
    g1 = _gat(xlt1, a1, row(att1), row(bc1), xr1, b1, cntt)
    h3, xlt2, xr2, a2, b2 = _prep2(
        h1, g1, Wp4, row(bp4),
        Wl2, bl2.reshape(HID, 1), Wr2, row(br2), row(att2))
    g2 = _gat(xlt2, a2, row(att2), row(bc2), xr2, b2, cntt)
    p, qt = _prep3(h3, g2, Wp5, row(bp5), We1, row(be1), We2, row(be2))
    return _edge(p, qt, row(We3[0]), be3.reshape(1, 1), keep)
